# baseline (speedup 1.0000x reference)
import jax
import jax.numpy as jnp
from jax import lax
from jax.experimental import pallas as pl
from jax.experimental.pallas import tpu as pltpu
from jax.experimental.pallas import tpu_sc as plsc

B = 16384
ED = 64
V = 1000000
CB = 2048
GRIDB = 245
K = CB * GRIDB
NC = 2
NS = 16
NW = NC * NS
BPW = B // NW
CHUNK = 128
NCHUNK = BPW // CHUNK
LANES = 16


def _pack_body(xa_ref, xb_ref, o_ref):
    ii = lax.broadcasted_iota(jnp.int32, (ED, ED), 0)
    jj = lax.broadcasted_iota(jnp.int32, (ED, ED), 1)
    eye = (ii == jj).astype(jnp.float32)
    dn0 = (((0,), (0,)), ((), ()))
    o_ref[:, 0:64] = lax.dot_general(
        xa_ref[...], eye, dn0, preferred_element_type=jnp.float32
    ).astype(jnp.bfloat16)
    o_ref[:, 64:128] = lax.dot_general(
        xb_ref[...], eye, dn0, preferred_element_type=jnp.float32
    ).astype(jnp.bfloat16)


def _pack(tabT):
    return pl.pallas_call(
        _pack_body,
        grid=(GRIDB,),
        in_specs=[
            pl.BlockSpec((ED, CB), lambda i: (0, i)),
            pl.BlockSpec((ED, CB), lambda i: (0, jnp.minimum(i + GRIDB, V // CB))),
        ],
        out_specs=pl.BlockSpec((CB, 128), lambda i: (i, 0)),
        out_shape=jax.ShapeDtypeStruct((K, 128), jnp.bfloat16),
    )(tabT, tabT)


def _rowmap_body(uid_ref, o_ref):
    u = uid_ref[...]
    o_ref[...] = jnp.where(u >= K, u - K, u)


def _rowmap(uid):
    return pl.pallas_call(
        _rowmap_body,
        grid=(1,),
        in_specs=[pl.BlockSpec((B, 1), lambda i: (0, 0))],
        out_specs=pl.BlockSpec((B, 1), lambda i: (0, 0)),
        out_shape=jax.ShapeDtypeStruct((B, 1), jnp.int32),
    )(uid)


def _user_gather_body(idx_hbm, t2_hbm, out_hbm, idx_v, rows_v, sem):
    wid = lax.axis_index("s") * NC + lax.axis_index("c")
    base = wid * NCHUNK
    pltpu.sync_copy(idx_hbm.at[pl.ds(base, NCHUNK)], idx_v)
    descs = []
    for j in range(NCHUNK):
        descs.append(
            pltpu.async_copy(
                t2_hbm.at[idx_v.at[j]],
                rows_v.at[pl.ds(j * CHUNK, CHUNK)],
                sem,
            )
        )
    for d in descs:
        d.wait()
    pltpu.sync_copy(rows_v, out_hbm.at[pl.ds(wid * BPW, BPW)])


def _user_gather(idx2d, t2):
    mesh = plsc.VectorSubcoreMesh(core_axis_name="c", subcore_axis_name="s")
    return pl.kernel(
        _user_gather_body,
        mesh=mesh,
        out_type=jax.ShapeDtypeStruct((B, 128), jnp.bfloat16),
        scratch_types=[
            pltpu.VMEM((NCHUNK, CHUNK), jnp.int32),
            pltpu.VMEM((BPW, 128), jnp.bfloat16),
            pltpu.SemaphoreType.DMA,
        ],
        compiler_params=pltpu.CompilerParams(use_tc_tiling_on_sc=False),
    )(idx2d, t2)


BLK = 2048
AGE_PAD = 104
GEN_PAD = 8


def _mlp_body(ue2_ref, uid_ref, age_ref, gen_ref, at_ref, gt_ref,
              w1_ref, b1_ref, w2_ref, b2_ref, w3_ref, b3_ref, o_ref):
    f32 = jnp.float32
    ue2 = ue2_ref[...].astype(jnp.float32)
    uid = uid_ref[...]
    age = age_ref[...]
    gen = gen_ref[...]

    half = (uid >= K).astype(jnp.int32)
    lane = lax.broadcasted_iota(jnp.int32, (BLK, 128), 1)
    keep = (lane // 64) == half
    uex = jnp.where(keep, ue2, 0.0)
    ue = uex[:, 0:64] + uex[:, 64:128]

    a_iota = lax.broadcasted_iota(jnp.int32, (BLK, AGE_PAD), 1)
    aoh = (age == a_iota).astype(f32)
    ae = jnp.dot(aoh, at_ref[...], preferred_element_type=f32)

    g_iota = lax.broadcasted_iota(jnp.int32, (BLK, GEN_PAD), 1)
    goh = (gen == g_iota).astype(f32)
    ge = jnp.dot(goh, gt_ref[...], preferred_element_type=f32)

    h = (jnp.dot(ue, w1_ref[0:64, :], preferred_element_type=f32)
         + jnp.dot(ae, w1_ref[64:96, :], preferred_element_type=f32)
         + jnp.dot(ge, w1_ref[96:112, :], preferred_element_type=f32)
         + b1_ref[...])
    h = jnp.maximum(h, 0.0)
    h = jnp.maximum(jnp.dot(h, w2_ref[...], preferred_element_type=f32)
                    + b2_ref[...], 0.0)
    v = jnp.dot(h, w3_ref[...], preferred_element_type=f32) + b3_ref[...]
    ss = jnp.sum(v * v, axis=1, keepdims=True)
    o_ref[...] = v / jnp.maximum(jnp.sqrt(ss), 1e-12)


def _mlp(ue2, uid, age, gen, at_pad, gt_pad, W1, b1, W2, b2, W3, b3):
    grid = (B // BLK,)
    const = lambda i: (0, 0)
    return pl.pallas_call(
        _mlp_body,
        grid=grid,
        in_specs=[
            pl.BlockSpec((BLK, 128), lambda i: (i, 0)),
            pl.BlockSpec((BLK, 1), lambda i: (i, 0)),
            pl.BlockSpec((BLK, 1), lambda i: (i, 0)),
            pl.BlockSpec((BLK, 1), lambda i: (i, 0)),
            pl.BlockSpec((AGE_PAD, 32), const),
            pl.BlockSpec((GEN_PAD, 16), const),
            pl.BlockSpec((112, 128), const),
            pl.BlockSpec((1, 128), const),
            pl.BlockSpec((128, 64), const),
            pl.BlockSpec((1, 64), const),
            pl.BlockSpec((64, 64), const),
            pl.BlockSpec((1, 64), const),
        ],
        out_specs=pl.BlockSpec((BLK, 64), lambda i: (i, 0)),
        out_shape=jax.ShapeDtypeStruct((B, 64), jnp.float32),
    )(ue2, uid, age, gen, at_pad, gt_pad, W1, b1, W2, b2, W3, b3)


def kernel(user_id, user_age, user_gender, user_table, age_table, gender_table,
           W1, b1, W2, b2, W3, b3):
    t2 = _pack(user_table.T)
    rows = _rowmap(user_id)
    ue2 = _user_gather(rows.reshape(B // CHUNK, CHUNK), t2)
    at_pad = jnp.pad(age_table, ((0, AGE_PAD - age_table.shape[0]), (0, 0)))
    gt_pad = jnp.pad(gender_table, ((0, GEN_PAD - gender_table.shape[0]), (0, 0)))
    return _mlp(ue2, user_id, user_age, user_gender, at_pad, gt_pad,
                W1, b1.reshape(1, -1), W2, b2.reshape(1, -1),
                W3, b3.reshape(1, -1))


# speedup vs baseline: 2.3839x; 2.3839x over previous
import jax
import jax.numpy as jnp
from jax import lax
from jax.experimental import pallas as pl
from jax.experimental.pallas import tpu as pltpu
from jax.experimental.pallas import tpu_sc as plsc

B = 16384
ED = 64
V = 1000000
CB = 4096
GRIDB = 123
K = CB * GRIDB
NC = 2
NS = 16
NW = NC * NS
BPW = B // NW
CHUNK = 128
NCHUNK = BPW // CHUNK
LANES = 16


def _pack_body(xa_ref, xb_ref, o_ref):
    ii = lax.broadcasted_iota(jnp.int32, (ED, ED), 0)
    jj = lax.broadcasted_iota(jnp.int32, (ED, ED), 1)
    eye = (ii == jj).astype(jnp.float32)
    dn0 = (((0,), (0,)), ((), ()))
    o_ref[:, 0:64] = lax.dot_general(xa_ref[...], eye, dn0,
                                     preferred_element_type=jnp.float32)
    o_ref[:, 64:128] = lax.dot_general(xb_ref[...], eye, dn0,
                                       preferred_element_type=jnp.float32)


def _pack(tabT):
    return pl.pallas_call(
        _pack_body,
        grid=(GRIDB,),
        in_specs=[
            pl.BlockSpec((ED, CB), lambda i: (0, i)),
            pl.BlockSpec((ED, CB), lambda i: (0, jnp.minimum(i + GRIDB, V // CB))),
        ],
        out_specs=pl.BlockSpec((CB, 128), lambda i: (i, 0)),
        out_shape=jax.ShapeDtypeStruct((K, 128), jnp.float32),
    )(tabT, tabT)


def _rowmap_body(uid_ref, o_ref):
    u = uid_ref[...]
    o_ref[...] = jnp.where(u >= K, u - K, u)


def _rowmap(uid):
    return pl.pallas_call(
        _rowmap_body,
        grid=(1,),
        in_specs=[pl.BlockSpec((B, 1), lambda i: (0, 0))],
        out_specs=pl.BlockSpec((B, 1), lambda i: (0, 0)),
        out_shape=jax.ShapeDtypeStruct((B, 1), jnp.int32),
    )(uid)


def _user_gather_body(idx_hbm, t2_hbm, out_hbm, idx_v, rows_v, sem):
    wid = lax.axis_index("s") * NC + lax.axis_index("c")
    base = wid * NCHUNK
    pltpu.sync_copy(idx_hbm.at[pl.ds(base, NCHUNK)], idx_v)
    descs = []
    for j in range(NCHUNK):
        descs.append(
            pltpu.async_copy(
                t2_hbm.at[idx_v.at[j]],
                rows_v.at[pl.ds(j * CHUNK, CHUNK)],
                sem,
            )
        )
    for d in descs:
        d.wait()
    pltpu.sync_copy(rows_v, out_hbm.at[pl.ds(wid * BPW, BPW)])


def _user_gather(idx2d, t2):
    mesh = plsc.VectorSubcoreMesh(core_axis_name="c", subcore_axis_name="s")
    return pl.kernel(
        _user_gather_body,
        mesh=mesh,
        out_type=jax.ShapeDtypeStruct((B, 128), jnp.float32),
        scratch_types=[
            pltpu.VMEM((NCHUNK, CHUNK), jnp.int32),
            pltpu.VMEM((BPW, 128), jnp.float32),
            pltpu.SemaphoreType.DMA,
        ],
        compiler_params=pltpu.CompilerParams(use_tc_tiling_on_sc=False),
    )(idx2d, t2)


BLK = 2048
AGE_PAD = 104
GEN_PAD = 8


def _mlp_body(ue2_ref, uid_ref, age_ref, gen_ref, at_ref, gt_ref,
              w1_ref, b1_ref, w2_ref, b2_ref, w3_ref, b3_ref, o_ref):
    f32 = jnp.float32
    ue2 = ue2_ref[...]
    uid = uid_ref[...]
    age = age_ref[...]
    gen = gen_ref[...]

    half = (uid >= K).astype(jnp.int32)
    lane = lax.broadcasted_iota(jnp.int32, (BLK, 128), 1)
    keep = (lane // 64) == half
    uex = jnp.where(keep, ue2, 0.0)
    ue = uex[:, 0:64] + uex[:, 64:128]

    a_iota = lax.broadcasted_iota(jnp.int32, (BLK, AGE_PAD), 1)
    aoh = (age == a_iota).astype(f32)
    ae = jnp.dot(aoh, at_ref[...], preferred_element_type=f32)

    g_iota = lax.broadcasted_iota(jnp.int32, (BLK, GEN_PAD), 1)
    goh = (gen == g_iota).astype(f32)
    ge = jnp.dot(goh, gt_ref[...], preferred_element_type=f32)

    h = (jnp.dot(ue, w1_ref[0:64, :], preferred_element_type=f32)
         + jnp.dot(ae, w1_ref[64:96, :], preferred_element_type=f32)
         + jnp.dot(ge, w1_ref[96:112, :], preferred_element_type=f32)
         + b1_ref[...])
    h = jnp.maximum(h, 0.0)
    h = jnp.maximum(jnp.dot(h, w2_ref[...], preferred_element_type=f32)
                    + b2_ref[...], 0.0)
    v = jnp.dot(h, w3_ref[...], preferred_element_type=f32) + b3_ref[...]
    ss = jnp.sum(v * v, axis=1, keepdims=True)
    o_ref[...] = v / jnp.maximum(jnp.sqrt(ss), 1e-12)


def _mlp(ue2, uid, age, gen, at_pad, gt_pad, W1, b1, W2, b2, W3, b3):
    grid = (B // BLK,)
    const = lambda i: (0, 0)
    return pl.pallas_call(
        _mlp_body,
        grid=grid,
        in_specs=[
            pl.BlockSpec((BLK, 128), lambda i: (i, 0)),
            pl.BlockSpec((BLK, 1), lambda i: (i, 0)),
            pl.BlockSpec((BLK, 1), lambda i: (i, 0)),
            pl.BlockSpec((BLK, 1), lambda i: (i, 0)),
            pl.BlockSpec((AGE_PAD, 32), const),
            pl.BlockSpec((GEN_PAD, 16), const),
            pl.BlockSpec((112, 128), const),
            pl.BlockSpec((1, 128), const),
            pl.BlockSpec((128, 64), const),
            pl.BlockSpec((1, 64), const),
            pl.BlockSpec((64, 64), const),
            pl.BlockSpec((1, 64), const),
        ],
        out_specs=pl.BlockSpec((BLK, 64), lambda i: (i, 0)),
        out_shape=jax.ShapeDtypeStruct((B, 64), jnp.float32),
    )(ue2, uid, age, gen, at_pad, gt_pad, W1, b1, W2, b2, W3, b3)


def kernel(user_id, user_age, user_gender, user_table, age_table, gender_table,
           W1, b1, W2, b2, W3, b3):
    t2 = _pack(user_table.T)
    rows = _rowmap(user_id)
    ue2 = _user_gather(rows.reshape(B // CHUNK, CHUNK), t2)
    at_pad = jnp.pad(age_table, ((0, AGE_PAD - age_table.shape[0]), (0, 0)))
    gt_pad = jnp.pad(gender_table, ((0, GEN_PAD - gender_table.shape[0]), (0, 0)))
    return _mlp(ue2, user_id, user_age, user_gender, at_pad, gt_pad,
                W1, b1.reshape(1, -1), W2, b2.reshape(1, -1),
                W3, b3.reshape(1, -1))


# speedup vs baseline: 2.6543x; 1.1134x over previous
import jax
import jax.numpy as jnp
from jax import lax
from jax.experimental import pallas as pl
from jax.experimental.pallas import tpu as pltpu
from jax.experimental.pallas import tpu_sc as plsc

B = 16384
ED = 64
V = 1000000
CB = 8192
GRIDB = 62
K = CB * GRIDB
NC = 2
NS = 16
NW = NC * NS
BPW = B // NW
CHUNK = 128
NCHUNK = BPW // CHUNK
LANES = 16


def _pack_body(xa_ref, xb_ref, o_ref):
    ii = lax.broadcasted_iota(jnp.int32, (ED, ED), 0)
    jj = lax.broadcasted_iota(jnp.int32, (ED, ED), 1)
    eye = (ii == jj).astype(jnp.float32)
    dn0 = (((0,), (0,)), ((), ()))
    o_ref[:, 0:64] = lax.dot_general(xa_ref[...], eye, dn0,
                                     preferred_element_type=jnp.float32)
    o_ref[:, 64:128] = lax.dot_general(xb_ref[...], eye, dn0,
                                       preferred_element_type=jnp.float32)


def _pack(tabT):
    return pl.pallas_call(
        _pack_body,
        grid=(GRIDB,),
        in_specs=[
            pl.BlockSpec((ED, CB), lambda i: (0, i)),
            pl.BlockSpec((ED, CB), lambda i: (0, jnp.minimum(i + GRIDB, V // CB))),
        ],
        out_specs=pl.BlockSpec((CB, 128), lambda i: (i, 0)),
        out_shape=jax.ShapeDtypeStruct((K, 128), jnp.float32),
    )(tabT, tabT)


def _rowmap_body(uid_ref, o_ref):
    u = uid_ref[...]
    o_ref[...] = jnp.where(u >= K, u - K, u)


def _rowmap(uid):
    return pl.pallas_call(
        _rowmap_body,
        grid=(1,),
        in_specs=[pl.BlockSpec((B, 1), lambda i: (0, 0))],
        out_specs=pl.BlockSpec((B, 1), lambda i: (0, 0)),
        out_shape=jax.ShapeDtypeStruct((B, 1), jnp.int32),
    )(uid)


def _user_gather_body(idx_hbm, t2_hbm, out_hbm, idx_v, rows_v, sem):
    wid = lax.axis_index("s") * NC + lax.axis_index("c")
    base = wid * NCHUNK
    pltpu.sync_copy(idx_hbm.at[pl.ds(base, NCHUNK)], idx_v)
    descs = []
    for j in range(NCHUNK):
        descs.append(
            pltpu.async_copy(
                t2_hbm.at[idx_v.at[j]],
                rows_v.at[pl.ds(j * CHUNK, CHUNK)],
                sem,
            )
        )
    for d in descs:
        d.wait()
    pltpu.sync_copy(rows_v, out_hbm.at[pl.ds(wid * BPW, BPW)])


def _user_gather(idx2d, t2):
    mesh = plsc.VectorSubcoreMesh(core_axis_name="c", subcore_axis_name="s")
    return pl.kernel(
        _user_gather_body,
        mesh=mesh,
        out_type=jax.ShapeDtypeStruct((B, 128), jnp.float32),
        scratch_types=[
            pltpu.VMEM((NCHUNK, CHUNK), jnp.int32),
            pltpu.VMEM((BPW, 128), jnp.float32),
            pltpu.SemaphoreType.DMA,
        ],
        compiler_params=pltpu.CompilerParams(use_tc_tiling_on_sc=False),
    )(idx2d, t2)


BLK = 2048
AGE_PAD = 104
GEN_PAD = 8


def _mlp_body(ue2_ref, uid_ref, age_ref, gen_ref, at_ref, gt_ref,
              w1_ref, b1_ref, w2_ref, b2_ref, w3_ref, b3_ref, o_ref):
    f32 = jnp.float32
    ue2 = ue2_ref[...]
    uid = uid_ref[...]
    age = age_ref[...]
    gen = gen_ref[...]

    half = (uid >= K).astype(jnp.int32)
    lane = lax.broadcasted_iota(jnp.int32, (BLK, 128), 1)
    keep = (lane // 64) == half
    uex = jnp.where(keep, ue2, 0.0)
    ue = uex[:, 0:64] + uex[:, 64:128]

    a_iota = lax.broadcasted_iota(jnp.int32, (BLK, AGE_PAD), 1)
    aoh = (age == a_iota).astype(f32)
    ae = jnp.dot(aoh, at_ref[...], preferred_element_type=f32)

    g_iota = lax.broadcasted_iota(jnp.int32, (BLK, GEN_PAD), 1)
    goh = (gen == g_iota).astype(f32)
    ge = jnp.dot(goh, gt_ref[...], preferred_element_type=f32)

    h = (jnp.dot(ue, w1_ref[0:64, :], preferred_element_type=f32)
         + jnp.dot(ae, w1_ref[64:96, :], preferred_element_type=f32)
         + jnp.dot(ge, w1_ref[96:112, :], preferred_element_type=f32)
         + b1_ref[...])
    h = jnp.maximum(h, 0.0)
    h = jnp.maximum(jnp.dot(h, w2_ref[...], preferred_element_type=f32)
                    + b2_ref[...], 0.0)
    v = jnp.dot(h, w3_ref[...], preferred_element_type=f32) + b3_ref[...]
    ss = jnp.sum(v * v, axis=1, keepdims=True)
    o_ref[...] = v / jnp.maximum(jnp.sqrt(ss), 1e-12)


def _mlp(ue2, uid, age, gen, at_pad, gt_pad, W1, b1, W2, b2, W3, b3):
    grid = (B // BLK,)
    const = lambda i: (0, 0)
    return pl.pallas_call(
        _mlp_body,
        grid=grid,
        in_specs=[
            pl.BlockSpec((BLK, 128), lambda i: (i, 0)),
            pl.BlockSpec((BLK, 1), lambda i: (i, 0)),
            pl.BlockSpec((BLK, 1), lambda i: (i, 0)),
            pl.BlockSpec((BLK, 1), lambda i: (i, 0)),
            pl.BlockSpec((AGE_PAD, 32), const),
            pl.BlockSpec((GEN_PAD, 16), const),
            pl.BlockSpec((112, 128), const),
            pl.BlockSpec((1, 128), const),
            pl.BlockSpec((128, 64), const),
            pl.BlockSpec((1, 64), const),
            pl.BlockSpec((64, 64), const),
            pl.BlockSpec((1, 64), const),
        ],
        out_specs=pl.BlockSpec((BLK, 64), lambda i: (i, 0)),
        out_shape=jax.ShapeDtypeStruct((B, 64), jnp.float32),
    )(ue2, uid, age, gen, at_pad, gt_pad, W1, b1, W2, b2, W3, b3)


def kernel(user_id, user_age, user_gender, user_table, age_table, gender_table,
           W1, b1, W2, b2, W3, b3):
    t2 = _pack(user_table.T)
    rows = _rowmap(user_id)
    ue2 = _user_gather(rows.reshape(B // CHUNK, CHUNK), t2)
    at_pad = jnp.pad(age_table, ((0, AGE_PAD - age_table.shape[0]), (0, 0)))
    gt_pad = jnp.pad(gender_table, ((0, GEN_PAD - gender_table.shape[0]), (0, 0)))
    return _mlp(ue2, user_id, user_age, user_gender, at_pad, gt_pad,
                W1, b1.reshape(1, -1), W2, b2.reshape(1, -1),
                W3, b3.reshape(1, -1))


# speedup vs baseline: 2.7646x; 1.0416x over previous
import jax
import jax.numpy as jnp
from jax import lax
from jax.experimental import pallas as pl
from jax.experimental.pallas import tpu as pltpu
from jax.experimental.pallas import tpu_sc as plsc

B = 16384
ED = 64
V = 1000000
CB = 16384
GRIDB = 31
K = CB * GRIDB
NC = 2
NS = 16
NW = NC * NS
BPW = B // NW
CHUNK = 128
NCHUNK = BPW // CHUNK
LANES = 16


def _pack_body(xa_ref, xb_ref, o_ref):
    ii = lax.broadcasted_iota(jnp.int32, (ED, ED), 0)
    jj = lax.broadcasted_iota(jnp.int32, (ED, ED), 1)
    eye = (ii == jj).astype(jnp.float32)
    dn0 = (((0,), (0,)), ((), ()))
    o_ref[:, 0:64] = lax.dot_general(xa_ref[...], eye, dn0,
                                     preferred_element_type=jnp.float32)
    o_ref[:, 64:128] = lax.dot_general(xb_ref[...], eye, dn0,
                                       preferred_element_type=jnp.float32)


def _pack(tabT):
    return pl.pallas_call(
        _pack_body,
        grid=(GRIDB,),
        in_specs=[
            pl.BlockSpec((ED, CB), lambda i: (0, i)),
            pl.BlockSpec((ED, CB), lambda i: (0, jnp.minimum(i + GRIDB, V // CB))),
        ],
        out_specs=pl.BlockSpec((CB, 128), lambda i: (i, 0)),
        out_shape=jax.ShapeDtypeStruct((K, 128), jnp.float32),
    )(tabT, tabT)


def _rowmap_body(uid_ref, o_ref):
    u = uid_ref[...]
    o_ref[...] = jnp.where(u >= K, u - K, u)


def _rowmap(uid):
    return pl.pallas_call(
        _rowmap_body,
        grid=(1,),
        in_specs=[pl.BlockSpec((B, 1), lambda i: (0, 0))],
        out_specs=pl.BlockSpec((B, 1), lambda i: (0, 0)),
        out_shape=jax.ShapeDtypeStruct((B, 1), jnp.int32),
    )(uid)


def _user_gather_body(idx_hbm, t2_hbm, out_hbm, idx_v, rows_v, sem):
    wid = lax.axis_index("s") * NC + lax.axis_index("c")
    base = wid * NCHUNK
    pltpu.sync_copy(idx_hbm.at[pl.ds(base, NCHUNK)], idx_v)
    descs = []
    for j in range(NCHUNK):
        descs.append(
            pltpu.async_copy(
                t2_hbm.at[idx_v.at[j]],
                rows_v.at[pl.ds(j * CHUNK, CHUNK)],
                sem,
            )
        )
    for d in descs:
        d.wait()
    pltpu.sync_copy(rows_v, out_hbm.at[pl.ds(wid * BPW, BPW)])


def _user_gather(idx2d, t2):
    mesh = plsc.VectorSubcoreMesh(core_axis_name="c", subcore_axis_name="s")
    return pl.kernel(
        _user_gather_body,
        mesh=mesh,
        out_type=jax.ShapeDtypeStruct((B, 128), jnp.float32),
        scratch_types=[
            pltpu.VMEM((NCHUNK, CHUNK), jnp.int32),
            pltpu.VMEM((BPW, 128), jnp.float32),
            pltpu.SemaphoreType.DMA,
        ],
        compiler_params=pltpu.CompilerParams(use_tc_tiling_on_sc=False),
    )(idx2d, t2)


BLK = 2048
AGE_PAD = 104
GEN_PAD = 8


def _mlp_body(ue2_ref, uid_ref, age_ref, gen_ref, at_ref, gt_ref,
              w1_ref, b1_ref, w2_ref, b2_ref, w3_ref, b3_ref, o_ref):
    f32 = jnp.float32
    ue2 = ue2_ref[...]
    uid = uid_ref[...]
    age = age_ref[...]
    gen = gen_ref[...]

    half = (uid >= K).astype(jnp.int32)
    lane = lax.broadcasted_iota(jnp.int32, (BLK, 128), 1)
    keep = (lane // 64) == half
    uex = jnp.where(keep, ue2, 0.0)
    ue = uex[:, 0:64] + uex[:, 64:128]

    a_iota = lax.broadcasted_iota(jnp.int32, (BLK, AGE_PAD), 1)
    aoh = (age == a_iota).astype(f32)
    ae = jnp.dot(aoh, at_ref[...], preferred_element_type=f32)

    g_iota = lax.broadcasted_iota(jnp.int32, (BLK, GEN_PAD), 1)
    goh = (gen == g_iota).astype(f32)
    ge = jnp.dot(goh, gt_ref[...], preferred_element_type=f32)

    h = (jnp.dot(ue, w1_ref[0:64, :], preferred_element_type=f32)
         + jnp.dot(ae, w1_ref[64:96, :], preferred_element_type=f32)
         + jnp.dot(ge, w1_ref[96:112, :], preferred_element_type=f32)
         + b1_ref[...])
    h = jnp.maximum(h, 0.0)
    h = jnp.maximum(jnp.dot(h, w2_ref[...], preferred_element_type=f32)
                    + b2_ref[...], 0.0)
    v = jnp.dot(h, w3_ref[...], preferred_element_type=f32) + b3_ref[...]
    ss = jnp.sum(v * v, axis=1, keepdims=True)
    o_ref[...] = v / jnp.maximum(jnp.sqrt(ss), 1e-12)


def _mlp(ue2, uid, age, gen, at_pad, gt_pad, W1, b1, W2, b2, W3, b3):
    grid = (B // BLK,)
    const = lambda i: (0, 0)
    return pl.pallas_call(
        _mlp_body,
        grid=grid,
        in_specs=[
            pl.BlockSpec((BLK, 128), lambda i: (i, 0)),
            pl.BlockSpec((BLK, 1), lambda i: (i, 0)),
            pl.BlockSpec((BLK, 1), lambda i: (i, 0)),
            pl.BlockSpec((BLK, 1), lambda i: (i, 0)),
            pl.BlockSpec((AGE_PAD, 32), const),
            pl.BlockSpec((GEN_PAD, 16), const),
            pl.BlockSpec((112, 128), const),
            pl.BlockSpec((1, 128), const),
            pl.BlockSpec((128, 64), const),
            pl.BlockSpec((1, 64), const),
            pl.BlockSpec((64, 64), const),
            pl.BlockSpec((1, 64), const),
        ],
        out_specs=pl.BlockSpec((BLK, 64), lambda i: (i, 0)),
        out_shape=jax.ShapeDtypeStruct((B, 64), jnp.float32),
    )(ue2, uid, age, gen, at_pad, gt_pad, W1, b1, W2, b2, W3, b3)


def kernel(user_id, user_age, user_gender, user_table, age_table, gender_table,
           W1, b1, W2, b2, W3, b3):
    t2 = _pack(user_table.T)
    rows = _rowmap(user_id)
    ue2 = _user_gather(rows.reshape(B // CHUNK, CHUNK), t2)
    at_pad = jnp.pad(age_table, ((0, AGE_PAD - age_table.shape[0]), (0, 0)))
    gt_pad = jnp.pad(gender_table, ((0, GEN_PAD - gender_table.shape[0]), (0, 0)))
    return _mlp(ue2, user_id, user_age, user_gender, at_pad, gt_pad,
                W1, b1.reshape(1, -1), W2, b2.reshape(1, -1),
                W3, b3.reshape(1, -1))
